# baseline (speedup 1.0000x reference)
import functools

import jax
import jax.numpy as jnp
from jax import lax
from jax.experimental import pallas as pl
from jax.experimental.pallas import tpu as pltpu
from jax.experimental.pallas import tpu_sc as plsc

CROP_RATIO = 0.8


@functools.lru_cache(maxsize=None)
def _crop_call(B, L, C, crop_len):
    info = plsc.get_sparse_core_info()
    NC, NS, NL = info.num_cores, info.num_subcores, info.num_lanes
    NW = NC * NS
    assert B == NW, "one subcore per batch element"
    assert B % NL == 0

    CHUNK = 128
    nch = crop_len // CHUNK
    rem = crop_len % CHUNK
    rem_pad = -(-rem // 8) * 8
    IDX = -(-max(crop_len, nch * CHUNK + rem_pad) // NL) * NL
    NBUF = 3

    mesh = plsc.VectorSubcoreMesh(core_axis_name="c", subcore_axis_name="s")

    @functools.partial(
        pl.kernel,
        mesh=mesh,
        compiler_params=pltpu.CompilerParams(
            needs_layout_passes=False, has_side_effects=True
        ),
        out_type=jax.ShapeDtypeStruct((B, crop_len, C), jnp.float32),
        scratch_types=[
            pltpu.VMEM((B,), jnp.int32),
            pltpu.VMEM((IDX,), jnp.int32),
        ]
        + [pltpu.VMEM((CHUNK, C), jnp.float32) for _ in range(NBUF)]
        + ([pltpu.VMEM((rem_pad, C), jnp.float32)] if rem else [])
        + [pltpu.SemaphoreType.DMA for _ in range(2 * NBUF + 2)],
    )
    def k(x_hbm, start_hbm, out_hbm, start_v, idx_v, *bufs_sems):
        bufs = bufs_sems[:NBUF]
        nb = NBUF + (1 if rem else 0)
        buf_last = bufs_sems[NBUF] if rem else None
        rsems = bufs_sems[nb : nb + NBUF + 1]
        wsems = bufs_sems[nb + NBUF + 1 :]
        wid = lax.axis_index("c") * NS + lax.axis_index("s")
        pltpu.sync_copy(start_hbm, start_v)

        lane = lax.iota(jnp.int32, NL)
        group = jnp.where(wid < NL, start_v[pl.ds(0, NL)], start_v[pl.ds(NL, NL)])
        s_splat = jnp.take(group, jnp.full((NL,), wid % NL, jnp.int32))

        base = s_splat + lane
        for j in range(IDX // NL):
            idx_v[pl.ds(NL * j, NL)] = jnp.minimum(
                base + NL * j, s_splat + crop_len - 1
            )

        def rd(i):
            if i < nch:
                return pltpu.make_async_copy(
                    x_hbm.at[wid].at[idx_v.at[pl.ds(i * CHUNK, CHUNK)]],
                    bufs[i % NBUF],
                    rsems[i % NBUF],
                )
            return pltpu.make_async_copy(
                x_hbm.at[wid].at[idx_v.at[pl.ds(nch * CHUNK, rem_pad)]],
                buf_last,
                rsems[NBUF],
            )

        def wr(i):
            if i < nch:
                return pltpu.make_async_copy(
                    bufs[i % NBUF],
                    out_hbm.at[wid, pl.ds(i * CHUNK, CHUNK), :],
                    wsems[i % NBUF],
                )
            r8 = rem - rem % 8
            cps = []
            if r8:
                cps.append(
                    pltpu.make_async_copy(
                        buf_last.at[pl.ds(0, r8), :],
                        out_hbm.at[wid, pl.ds(nch * CHUNK, r8), :],
                        wsems[NBUF],
                    )
                )
            for c0 in range(0, C, 128):
                cps.append(
                    pltpu.make_async_copy(
                        buf_last.at[pl.ds(r8, rem - r8), pl.ds(c0, 128)],
                        out_hbm.at[
                            wid, pl.ds(nch * CHUNK + r8, rem - r8), pl.ds(c0, 128)
                        ],
                        wsems[NBUF],
                    )
                )
            return cps

        if rem:
            rd(nch).start()

        ahead = NBUF - 1
        for i in range(min(ahead, nch)):
            rd(i).start()
        for i in range(nch):
            rd(i).wait()
            wr(i).start()
            j = i + ahead
            if j < nch:
                if j - NBUF >= 0:
                    wr(j - NBUF).wait()
                rd(j).start()
        if rem:
            rd(nch).wait()
            for cp in wr(nch):
                cp.start()
        for i in range(max(0, nch - NBUF), nch):
            wr(i).wait()
        if rem:
            for cp in wr(nch):
                cp.wait()

    return k


def kernel(x):
    B, L, C = x.shape
    crop_len = int(L * CROP_RATIO)
    start = jax.random.randint(
        jax.random.key(1), (B,), 0, L - crop_len + 1
    ).astype(jnp.int32)
    return _crop_call(B, L, C, crop_len)(x, start)


# speedup vs baseline: 1.1049x; 1.1049x over previous
import functools

import jax
import jax.numpy as jnp
from jax import lax
from jax.experimental import pallas as pl
from jax.experimental.pallas import tpu as pltpu
from jax.experimental.pallas import tpu_sc as plsc

CROP_RATIO = 0.8


@functools.lru_cache(maxsize=None)
def _crop_call(B, L, C, crop_len):
    info = plsc.get_sparse_core_info()
    NC, NS, NL = info.num_cores, info.num_subcores, info.num_lanes
    NW = NC * NS
    assert B % NL == 0
    TCH = 128 // B
    assert crop_len % TCH == 0
    NCHT = crop_len // TCH
    q, extra = divmod(NCHT, NW)
    NJ = q + (1 if extra else 0)
    NBUF = 3
    ROWS = TCH * B
    PADC = (q * (NW - 1) + min(NW - 1, extra)) + NJ

    mesh = plsc.VectorSubcoreMesh(core_axis_name="c", subcore_axis_name="s")

    @functools.partial(
        pl.kernel,
        mesh=mesh,
        compiler_params=pltpu.CompilerParams(needs_layout_passes=False),
        out_type=jax.ShapeDtypeStruct((crop_len * B, C), jnp.float32),
        scratch_types=[
            pltpu.VMEM((NJ * ROWS,), jnp.int32),
        ]
        + [pltpu.VMEM((ROWS, C), jnp.float32) for _ in range(NBUF)]
        + [pltpu.SemaphoreType.DMA for _ in range(2 * NBUF)],
    )
    def k(x2_hbm, idx_hbm, out_hbm, idx_v, *bufs_sems):
        bufs = bufs_sems[:NBUF]
        rsems = bufs_sems[NBUF : 2 * NBUF]
        wsems = bufs_sems[2 * NBUF :]
        wid = lax.axis_index("c") * NS + lax.axis_index("s")

        c0 = q * wid + jnp.minimum(wid, extra)
        pltpu.sync_copy(
            idx_hbm.at[pl.ds(pl.multiple_of(c0 * ROWS, ROWS), NJ * ROWS)], idx_v
        )

        def tc_of(j):
            return jnp.minimum(c0 + j, NCHT - 1)

        def rd(j):
            return pltpu.make_async_copy(
                x2_hbm.at[idx_v.at[pl.ds(j * ROWS, ROWS)]],
                bufs[j % NBUF],
                rsems[j % NBUF],
            )

        def wr(j):
            return pltpu.make_async_copy(
                bufs[j % NBUF],
                out_hbm.at[pl.ds(pl.multiple_of(tc_of(j) * ROWS, ROWS), ROWS), :],
                wsems[j % NBUF],
            )

        ahead = NBUF - 1
        for j in range(min(ahead, NJ)):
            rd(j).start()
        for j in range(NJ):
            rd(j).wait()
            wr(j).start()
            nxt = j + ahead
            if nxt < NJ:
                if nxt - NBUF >= 0:
                    wr(nxt - NBUF).wait()
                rd(nxt).start()
        for j in range(max(0, NJ - NBUF), NJ):
            wr(j).wait()

    return k, PADC * ROWS


def kernel(x):
    B, L, C = x.shape
    crop_len = int(L * CROP_RATIO)
    start = jax.random.randint(
        jax.random.key(1), (B,), 0, L - crop_len + 1
    ).astype(jnp.int32)
    call, pad_rows = _crop_call(B, L, C, crop_len)
    tch = 128 // B
    p = jnp.arange(pad_rows, dtype=jnp.int32)
    b = p % B
    t = p // B
    t = jnp.where(t >= crop_len, t - tch, t)
    idx = b * L + jnp.take(start, b) + t
    out2 = call(x.reshape(B * L, C), idx)
    return out2.reshape(crop_len, B, C).transpose(1, 0, 2)


# speedup vs baseline: 1.7905x; 1.6206x over previous
import functools

import jax
import jax.numpy as jnp
from jax import lax
from jax.experimental import pallas as pl
from jax.experimental.pallas import tpu as pltpu
from jax.experimental.pallas import tpu_sc as plsc

CROP_RATIO = 0.8


@functools.lru_cache(maxsize=None)
def _crop_call(B, L, C, crop_len):
    info = plsc.get_sparse_core_info()
    NC, NS, NL = info.num_cores, info.num_subcores, info.num_lanes
    NW = NC * NS
    assert B % NL == 0
    TCH = 128 // B
    assert crop_len % TCH == 0
    NCHT = crop_len // TCH
    q, extra = divmod(NCHT, NW)
    NJ = q + (1 if extra else 0)
    NBUF = 3
    ROWS = TCH * B
    PADC = (q * (NW - 1) + min(NW - 1, extra)) + NJ

    mesh = plsc.VectorSubcoreMesh(core_axis_name="c", subcore_axis_name="s")

    @functools.partial(
        pl.kernel,
        mesh=mesh,
        compiler_params=pltpu.CompilerParams(needs_layout_passes=False),
        out_type=jax.ShapeDtypeStruct((crop_len * B, C), jnp.float32),
        scratch_types=[
            pltpu.VMEM((NJ * ROWS,), jnp.int32),
        ]
        + [pltpu.VMEM((ROWS, C), jnp.float32) for _ in range(NBUF)]
        + [pltpu.SemaphoreType.DMA for _ in range(2 * NBUF)],
    )
    def k(x2_hbm, idx_hbm, out_hbm, idx_v, *bufs_sems):
        bufs = bufs_sems[:NBUF]
        rsems = bufs_sems[NBUF : 2 * NBUF]
        wsems = bufs_sems[2 * NBUF :]
        wid = lax.axis_index("c") * NS + lax.axis_index("s")

        c0 = q * wid + jnp.minimum(wid, extra)
        pltpu.sync_copy(
            idx_hbm.at[pl.ds(pl.multiple_of(c0 * ROWS, ROWS), NJ * ROWS)], idx_v
        )

        def tc_of(j):
            return jnp.minimum(c0 + j, NCHT - 1)

        def rd(j):
            return pltpu.make_async_copy(
                x2_hbm.at[idx_v.at[pl.ds(j * ROWS, ROWS)]],
                bufs[j % NBUF],
                rsems[j % NBUF],
            )

        def wr(j):
            return pltpu.make_async_copy(
                bufs[j % NBUF],
                out_hbm.at[pl.ds(pl.multiple_of(tc_of(j) * ROWS, ROWS), ROWS), :],
                wsems[j % NBUF],
            )

        ahead = NBUF - 1
        for j in range(min(ahead, NJ)):
            rd(j).start()
        for j in range(NJ):
            rd(j).wait()
            wr(j).start()
            nxt = j + ahead
            if nxt < NJ:
                if nxt - NBUF >= 0:
                    wr(nxt - NBUF).wait()
                rd(nxt).start()
        for j in range(max(0, NJ - NBUF), NJ):
            wr(j).wait()

    return k, PADC * ROWS


def kernel(x):
    B, L, C = x.shape
    crop_len = int(L * CROP_RATIO)
    start = jax.random.randint(
        jax.random.key(1), (B,), 0, L - crop_len + 1
    ).astype(jnp.int32)
    call, pad_rows = _crop_call(B, L, C, crop_len)
    tch = 128 // B
    t = jnp.arange(pad_rows // B, dtype=jnp.int32)
    t = jnp.where(t >= crop_len, t - tch, t)
    g = jnp.arange(B, dtype=jnp.int32) * L + start
    idx = (g[None, :] + t[:, None]).reshape(-1)
    out2 = call(x.reshape(B * L, C), idx)
    return out2.reshape(crop_len, B, C).transpose(1, 0, 2)


# speedup vs baseline: 1.8511x; 1.0338x over previous
import functools

import jax
import jax.numpy as jnp
from jax import lax
from jax.experimental import pallas as pl
from jax.experimental.pallas import tpu as pltpu
from jax.experimental.pallas import tpu_sc as plsc

CROP_RATIO = 0.8


@functools.lru_cache(maxsize=None)
def _crop_call(B, L, C, crop_len):
    info = plsc.get_sparse_core_info()
    NC, NS, NL = info.num_cores, info.num_subcores, info.num_lanes
    NW = NC * NS
    assert B % NL == 0
    TCH = 128 // B
    assert crop_len % TCH == 0
    NCHT = crop_len // TCH
    q, extra = divmod(NCHT, NW)
    NJ = q + (1 if extra else 0)
    NBUF = 3
    ROWS = TCH * B
    PADC = (q * (NW - 1) + min(NW - 1, extra)) + NJ

    mesh = plsc.VectorSubcoreMesh(core_axis_name="c", subcore_axis_name="s")

    @functools.partial(
        pl.kernel,
        mesh=mesh,
        compiler_params=pltpu.CompilerParams(needs_layout_passes=False),
        out_type=jax.ShapeDtypeStruct((crop_len * B, C), jnp.float32),
        scratch_types=[
            pltpu.VMEM((NJ * ROWS,), jnp.int32),
        ]
        + [pltpu.VMEM((ROWS, C), jnp.float32) for _ in range(NBUF)]
        + [pltpu.SemaphoreType.DMA for _ in range(2 * NBUF)],
    )
    def k(x2_hbm, idx_hbm, out_hbm, idx_v, *bufs_sems):
        bufs = bufs_sems[:NBUF]
        rsems = bufs_sems[NBUF : 2 * NBUF]
        wsems = bufs_sems[2 * NBUF :]
        wid = lax.axis_index("c") * NS + lax.axis_index("s")

        c0 = q * wid + jnp.minimum(wid, extra)
        pltpu.sync_copy(
            idx_hbm.at[pl.ds(pl.multiple_of(c0 * ROWS, ROWS), NJ * ROWS)], idx_v
        )

        def tc_of(j):
            return jnp.minimum(c0 + j, NCHT - 1)

        def rd(j):
            return pltpu.make_async_copy(
                x2_hbm.at[idx_v.at[pl.ds(j * ROWS, ROWS)]],
                bufs[j % NBUF],
                rsems[j % NBUF],
            )

        def wr(j):
            return pltpu.make_async_copy(
                bufs[j % NBUF],
                out_hbm.at[pl.ds(pl.multiple_of(tc_of(j) * ROWS, ROWS), ROWS), :],
                wsems[j % NBUF],
            )

        ahead = NBUF - 1
        for j in range(min(ahead, NJ)):
            rd(j).start()
        for j in range(NJ):
            rd(j).wait()
            wr(j).start()
            nxt = j + ahead
            if nxt < NJ:
                if nxt - NBUF >= 0:
                    wr(nxt - NBUF).wait()
                rd(nxt).start()
        for j in range(max(0, NJ - NBUF), NJ):
            wr(j).wait()

    return k, PADC * ROWS


def kernel(x):
    B, L, C = x.shape
    crop_len = int(L * CROP_RATIO)
    call, pad_rows = _crop_call(B, L, C, crop_len)
    with jax.ensure_compile_time_eval():
        start = jax.random.randint(
            jax.random.key(1), (B,), 0, L - crop_len + 1
        ).astype(jnp.int32)
        tch = 128 // B
        t = jnp.arange(pad_rows // B, dtype=jnp.int32)
        t = jnp.where(t >= crop_len, t - tch, t)
        g = jnp.arange(B, dtype=jnp.int32) * L + start
        idx = jnp.asarray((g[None, :] + t[:, None]).reshape(-1))
    out2 = call(x.reshape(B * L, C), idx)
    return out2.reshape(crop_len, B, C).transpose(1, 0, 2)


# speedup vs baseline: 1.8846x; 1.0181x over previous
import functools

import jax
import jax.numpy as jnp
from jax import lax
from jax.experimental import pallas as pl
from jax.experimental.pallas import tpu as pltpu
from jax.experimental.pallas import tpu_sc as plsc

CROP_RATIO = 0.8


@functools.lru_cache(maxsize=None)
def _crop_call(B, L, C, crop_len):
    info = plsc.get_sparse_core_info()
    NC, NS, NL = info.num_cores, info.num_subcores, info.num_lanes
    NW = NC * NS
    assert B % NL == 0
    ROWS = 64
    TCH = ROWS // B
    assert crop_len % TCH == 0
    NCHT = crop_len // TCH
    q, extra = divmod(NCHT, NW)
    NJ = q + (1 if extra else 0)
    NBUF = 6
    PADC = (q * (NW - 1) + min(NW - 1, extra)) + NJ

    mesh = plsc.VectorSubcoreMesh(core_axis_name="c", subcore_axis_name="s")

    @functools.partial(
        pl.kernel,
        mesh=mesh,
        compiler_params=pltpu.CompilerParams(needs_layout_passes=False),
        out_type=jax.ShapeDtypeStruct((crop_len * B, C), jnp.float32),
        scratch_types=[
            pltpu.VMEM((NJ * ROWS,), jnp.int32),
        ]
        + [pltpu.VMEM((ROWS, C), jnp.float32) for _ in range(NBUF)]
        + [pltpu.SemaphoreType.DMA for _ in range(2 * NBUF)],
    )
    def k(x2_hbm, idx_hbm, out_hbm, idx_v, *bufs_sems):
        bufs = bufs_sems[:NBUF]
        rsems = bufs_sems[NBUF : 2 * NBUF]
        wsems = bufs_sems[2 * NBUF :]
        wid = lax.axis_index("c") * NS + lax.axis_index("s")

        c0 = q * wid + jnp.minimum(wid, extra)
        pltpu.sync_copy(
            idx_hbm.at[pl.ds(pl.multiple_of(c0 * ROWS, ROWS), NJ * ROWS)], idx_v
        )

        def tc_of(j):
            return jnp.minimum(c0 + j, NCHT - 1)

        def rd(j):
            return pltpu.make_async_copy(
                x2_hbm.at[idx_v.at[pl.ds(j * ROWS, ROWS)]],
                bufs[j % NBUF],
                rsems[j % NBUF],
            )

        def wr(j):
            return pltpu.make_async_copy(
                bufs[j % NBUF],
                out_hbm.at[pl.ds(pl.multiple_of(tc_of(j) * ROWS, ROWS), ROWS), :],
                wsems[j % NBUF],
            )

        ahead = NBUF - 1
        for j in range(min(ahead, NJ)):
            rd(j).start()
        for j in range(NJ):
            rd(j).wait()
            wr(j).start()
            nxt = j + ahead
            if nxt < NJ:
                if nxt - NBUF >= 0:
                    wr(nxt - NBUF).wait()
                rd(nxt).start()
        for j in range(max(0, NJ - NBUF), NJ):
            wr(j).wait()

    return k, PADC * ROWS, TCH


def kernel(x):
    B, L, C = x.shape
    crop_len = int(L * CROP_RATIO)
    call, pad_rows, tch = _crop_call(B, L, C, crop_len)
    with jax.ensure_compile_time_eval():
        start = jax.random.randint(
            jax.random.key(1), (B,), 0, L - crop_len + 1
        ).astype(jnp.int32)
        t = jnp.arange(pad_rows // B, dtype=jnp.int32)
        t = jnp.where(t >= crop_len, t - tch, t)
        g = jnp.arange(B, dtype=jnp.int32) * L + start
        idx = jnp.asarray((g[None, :] + t[:, None]).reshape(-1))
    out2 = call(x.reshape(B * L, C), idx)
    return out2.reshape(crop_len, B, C).transpose(1, 0, 2)


# speedup vs baseline: 1.8904x; 1.0031x over previous
import functools

import jax
import jax.numpy as jnp
from jax import lax
from jax.experimental import pallas as pl
from jax.experimental.pallas import tpu as pltpu
from jax.experimental.pallas import tpu_sc as plsc

CROP_RATIO = 0.8


@functools.lru_cache(maxsize=None)
def _crop_call(B, L, C, crop_len):
    info = plsc.get_sparse_core_info()
    NC, NS, NL = info.num_cores, info.num_subcores, info.num_lanes
    NW = NC * NS
    assert B % NL == 0
    ROWS = 64
    TCH = ROWS // B
    assert crop_len % TCH == 0
    NCHT = crop_len // TCH
    q, extra = divmod(NCHT, NW)
    NJ = q + (1 if extra else 0)
    NBUF = 7
    PADC = (q * (NW - 1) + min(NW - 1, extra)) + NJ

    mesh = plsc.VectorSubcoreMesh(core_axis_name="c", subcore_axis_name="s")

    @functools.partial(
        pl.kernel,
        mesh=mesh,
        compiler_params=pltpu.CompilerParams(needs_layout_passes=False),
        out_type=jax.ShapeDtypeStruct((crop_len * B, C), jnp.float32),
        scratch_types=[
            pltpu.VMEM((NJ * ROWS,), jnp.int32),
        ]
        + [pltpu.VMEM((ROWS, C), jnp.float32) for _ in range(NBUF)]
        + [pltpu.SemaphoreType.DMA for _ in range(2 * NBUF)],
    )
    def k(x2_hbm, idx_hbm, out_hbm, idx_v, *bufs_sems):
        bufs = bufs_sems[:NBUF]
        rsems = bufs_sems[NBUF : 2 * NBUF]
        wsems = bufs_sems[2 * NBUF :]
        wid = lax.axis_index("c") * NS + lax.axis_index("s")

        c0 = q * wid + jnp.minimum(wid, extra)
        pltpu.sync_copy(
            idx_hbm.at[pl.ds(pl.multiple_of(c0 * ROWS, ROWS), NJ * ROWS)], idx_v
        )

        def tc_of(j):
            return jnp.minimum(c0 + j, NCHT - 1)

        def rd(j):
            return pltpu.make_async_copy(
                x2_hbm.at[idx_v.at[pl.ds(j * ROWS, ROWS)]],
                bufs[j % NBUF],
                rsems[j % NBUF],
            )

        def wr(j):
            return pltpu.make_async_copy(
                bufs[j % NBUF],
                out_hbm.at[pl.ds(pl.multiple_of(tc_of(j) * ROWS, ROWS), ROWS), :],
                wsems[j % NBUF],
            )

        ahead = NBUF - 1
        for j in range(min(ahead, NJ)):
            rd(j).start()
        for j in range(NJ):
            rd(j).wait()
            wr(j).start()
            nxt = j + ahead
            if nxt < NJ:
                if nxt - NBUF >= 0:
                    wr(nxt - NBUF).wait()
                rd(nxt).start()
        for j in range(max(0, NJ - NBUF), NJ):
            wr(j).wait()

    return k, PADC * ROWS, TCH


def kernel(x):
    B, L, C = x.shape
    crop_len = int(L * CROP_RATIO)
    call, pad_rows, tch = _crop_call(B, L, C, crop_len)
    with jax.ensure_compile_time_eval():
        start = jax.random.randint(
            jax.random.key(1), (B,), 0, L - crop_len + 1
        ).astype(jnp.int32)
        t = jnp.arange(pad_rows // B, dtype=jnp.int32)
        t = jnp.where(t >= crop_len, t - tch, t)
        g = jnp.arange(B, dtype=jnp.int32) * L + start
        idx = jnp.asarray((g[None, :] + t[:, None]).reshape(-1))
    out2 = call(x.reshape(B * L, C), idx)
    return out2.reshape(crop_len, B, C).transpose(1, 0, 2)
